# baseline (speedup 1.0000x reference)
import functools

import jax
import jax.numpy as jnp
from jax import lax
from jax.experimental import pallas as pl
from jax.experimental.pallas import tpu as pltpu
from jax.experimental.pallas import tpu_sc as plsc

_NC = 2
_NS = 16
_NW = _NC * _NS
_PC = 16
_PG = 32
_NBUF = 4
_L = 16


@functools.lru_cache(maxsize=None)
def _build_sc_call(B, T, D):
    PW = T // _NW
    NPG = PW // _PG
    CPG = B * (_PG // _PC)
    NCH = NPG * CPG
    VPR = D // _L

    mesh = plsc.VectorSubcoreMesh(core_axis_name="c", subcore_axis_name="s")

    @functools.partial(
        pl.kernel,
        mesh=mesh,
        out_type=jax.ShapeDtypeStruct((B * T, D), jnp.float32),
        scratch_types=[
            pltpu.VMEM((NCH, _PC), jnp.int32),
            pltpu.VMEM((_PG, D), jnp.float32),
            pltpu.VMEM((_NBUF, _PC, D), jnp.float32),
            pltpu.SemaphoreType.DMA,
            pltpu.SemaphoreType.DMA,
            pltpu.SemaphoreType.DMA,
            pltpu.SemaphoreType.DMA,
            pltpu.SemaphoreType.DMA,
            pltpu.SemaphoreType.DMA,
            pltpu.SemaphoreType.DMA,
            pltpu.SemaphoreType.DMA,
            pltpu.SemaphoreType.DMA,
        ],
    )
    def sc_embed(idx_hbm, emb_hbm, pos_hbm, out_hbm, idx_v, pos_v, tok_v,
                 g0, g1, g2, g3, w0, w1, w2, w3, psem):
        gsems = [g0, g1, g2, g3]
        wsems = [w0, w1, w2, w3]
        wid = lax.axis_index("s") * _NC + lax.axis_index("c")
        pos0 = wid * PW

        pltpu.sync_copy(idx_hbm.at[wid], idx_v)

        def start_gather(row, buf):
            pltpu.async_copy(emb_hbm.at[idx_v.at[row]], tok_v.at[buf],
                             gsems[buf])

        def wait_gather(buf):
            pltpu.make_async_copy(emb_hbm.at[idx_v.at[0]], tok_v.at[buf],
                                  gsems[buf]).wait()

        def start_write(base, buf):
            pltpu.async_copy(tok_v.at[buf], out_hbm.at[pl.ds(base, _PC)],
                             wsems[buf])

        def wait_write(buf):
            pltpu.make_async_copy(tok_v.at[buf], out_hbm.at[pl.ds(0, _PC)],
                                  wsems[buf]).wait()

        def add_pos(buf, h):
            def row_body(r, carry):
                pr = h * _PC + r
                for v in range(VPR):
                    sl = pl.ds(v * _L, _L)
                    tok_v[buf, r, sl] = tok_v[buf, r, sl] + pos_v[pr, sl]
                return carry
            lax.fori_loop(0, _PC, row_body, 0)

        ph = pltpu.async_copy(pos_hbm.at[pl.ds(pos0, _PG)], pos_v, psem)
        for buf in range(_NBUF - 1):
            start_gather(buf, buf)
        ph.wait()
        for j in range(CPG):
            b, h = divmod(j, _PG // _PC)
            buf = j % _NBUF
            nk = j + _NBUF - 1
            nbuf = nk % _NBUF
            if nk >= _NBUF:
                wait_write(nbuf)
            start_gather(nk, nbuf)
            wait_gather(buf)
            add_pos(buf, h)
            start_write(b * T + pos0 + h * _PC, buf)

        def group_body(i, carry):
            base = (i + 1) * CPG
            ph2 = pltpu.async_copy(
                pos_hbm.at[pl.ds(pos0 + base * (_PG // CPG), _PG)], pos_v,
                psem)
            for j in range(CPG):
                b, h = divmod(j, _PG // _PC)
                buf = j % _NBUF
                nrow = jnp.minimum(base + j + _NBUF - 1, NCH - 1)
                nbuf = (j + _NBUF - 1) % _NBUF
                wait_write(nbuf)
                start_gather(nrow, nbuf)
                if j == 0:
                    ph2.wait()
                wait_gather(buf)
                add_pos(buf, h)
                start_write(b * T + pos0 + base * (_PG // CPG) + h * _PC, buf)
            return carry
        lax.fori_loop(0, NPG - 1, group_body, 0)

        wait_write((NCH - 1) % _NBUF)
        for extra in range(_NBUF - 1):
            wait_gather((NCH + extra) % _NBUF)

    return sc_embed


def kernel(token_ids, token_emb, pos_emb):
    B, T = token_ids.shape
    V, D = token_emb.shape
    PW = T // _NW
    NPG = PW // _PG
    HPG = _PG // _PC

    ids = token_ids.astype(jnp.int32)
    idx = (ids.reshape(B, _NW, NPG, HPG, _PC)
              .transpose(1, 2, 0, 3, 4)
              .reshape(_NW, NPG * B * HPG, _PC))

    sc_embed = _build_sc_call(B, T, D)
    out_flat = sc_embed(idx, token_emb, pos_emb)
    return out_flat.reshape(B, T, D)


# speedup vs baseline: 1.0809x; 1.0809x over previous
import functools

import jax
import jax.numpy as jnp
from jax import lax
from jax.experimental import pallas as pl
from jax.experimental.pallas import tpu as pltpu
from jax.experimental.pallas import tpu_sc as plsc

_NC = 2
_NS = 16
_NW = _NC * _NS
_PC = 16
_NBUF = 4
_PD = 2
_L = 16


@functools.lru_cache(maxsize=None)
def _build_sc_call(B, T, D):
    PW = T // _NW
    NPC = PW // _PC
    NCH = NPC * B
    VPR = D // _L
    PERIOD = 2 * B

    mesh = plsc.VectorSubcoreMesh(core_axis_name="c", subcore_axis_name="s")

    @functools.partial(
        pl.kernel,
        mesh=mesh,
        out_type=jax.ShapeDtypeStruct((B * T, D), jnp.float32),
        scratch_types=[
            pltpu.VMEM((NCH, _PC), jnp.int32),
            pltpu.VMEM((2, _PC, D), jnp.float32),
            pltpu.VMEM((_NBUF, _PC, D), jnp.float32),
            pltpu.SemaphoreType.DMA,
            pltpu.SemaphoreType.DMA,
            pltpu.SemaphoreType.DMA,
            pltpu.SemaphoreType.DMA,
            pltpu.SemaphoreType.DMA,
            pltpu.SemaphoreType.DMA,
            pltpu.SemaphoreType.DMA,
            pltpu.SemaphoreType.DMA,
            pltpu.SemaphoreType.DMA,
            pltpu.SemaphoreType.DMA,
        ],
    )
    def sc_embed(idx_hbm, emb_hbm, pos_hbm, out_hbm, idx_v, pos_v, tok_v,
                 g0, g1, g2, g3, w0, w1, w2, w3, p0, p1):
        gsems = [g0, g1, g2, g3]
        wsems = [w0, w1, w2, w3]
        psems = [p0, p1]
        wid = lax.axis_index("s") * _NC + lax.axis_index("c")
        pos0 = wid * PW

        pltpu.sync_copy(idx_hbm.at[wid], idx_v)

        def start_gather(row, buf):
            pltpu.async_copy(emb_hbm.at[idx_v.at[row]], tok_v.at[buf],
                             gsems[buf])

        def wait_gather(buf):
            pltpu.make_async_copy(emb_hbm.at[idx_v.at[0]], tok_v.at[buf],
                                  gsems[buf]).wait()

        def start_write(base, buf):
            pltpu.async_copy(tok_v.at[buf], out_hbm.at[pl.ds(base, _PC)],
                             wsems[buf])

        def wait_write(buf):
            pltpu.make_async_copy(tok_v.at[buf], out_hbm.at[pl.ds(0, _PC)],
                                  wsems[buf]).wait()

        def start_pos(p, pbuf):
            off = jnp.minimum(pos0 + p * _PC, T - _PC)
            pltpu.async_copy(pos_hbm.at[pl.ds(off, _PC)], pos_v.at[pbuf],
                             psems[pbuf])

        def wait_pos(pbuf):
            pltpu.make_async_copy(pos_hbm.at[pl.ds(0, _PC)], pos_v.at[pbuf],
                                  psems[pbuf]).wait()

        def add_pos(buf, pbuf):
            def row_body(r, carry):
                for v in range(VPR):
                    sl = pl.ds(v * _L, _L)
                    tok_v[buf, r, sl] = tok_v[buf, r, sl] + pos_v[pbuf, r, sl]
                return carry
            lax.fori_loop(0, _PC, row_body, 0)

        def chunk_body(k, j, static):
            b = j % B
            buf = j % _NBUF
            nbuf = (j + _PD) % _NBUF
            pj = (j // B) % 2
            if static:
                if k >= _NBUF - _PD:
                    wait_write(nbuf)
                if k + _PD < NCH:
                    start_gather(k + _PD, nbuf)
                if b == 0:
                    wait_pos(pj)
                    npos = k // B + 1
                    if npos < NPC:
                        start_pos(npos, npos % 2)
            else:
                wait_write(nbuf)
                start_gather(jnp.minimum(k + _PD, NCH - 1), nbuf)
                if b == 0:
                    wait_pos(pj)
                    start_pos(jnp.minimum(k // B + 1, NPC - 1),
                              (j // B + 1) % 2)
            wait_gather(buf)
            add_pos(buf, pj)
            start_write(b * T + pos0 + (k // B) * _PC, buf)

        start_pos(0, 0)
        for r in range(_PD):
            start_gather(r, r)
        for j in range(PERIOD):
            chunk_body(j, j, True)

        def period_body(i, carry):
            base = (i + 1) * PERIOD
            for j in range(PERIOD):
                chunk_body(base + j, j, False)
            return carry
        lax.fori_loop(0, NCH // PERIOD - 1, period_body, 0)

        for tail in range(_NBUF - _PD, 0, -1):
            wait_write((NCH - tail) % _NBUF)
        for extra in range(_PD):
            wait_gather((NCH + extra) % _NBUF)
        wait_pos(NPC % 2)

    return sc_embed


def kernel(token_ids, token_emb, pos_emb):
    B, T = token_ids.shape
    V, D = token_emb.shape
    PW = T // _NW
    NPC = PW // _PC

    ids = token_ids.astype(jnp.int32)
    idx = (ids.reshape(B, _NW, NPC, _PC)
              .transpose(1, 2, 0, 3)
              .reshape(_NW, NPC * B, _PC))

    sc_embed = _build_sc_call(B, T, D)
    out_flat = sc_embed(idx, token_emb, pos_emb)
    return out_flat.reshape(B, T, D)


# speedup vs baseline: 1.5869x; 1.4682x over previous
import functools

import jax
import jax.numpy as jnp
from jax import lax
from jax.experimental import pallas as pl
from jax.experimental.pallas import tpu as pltpu
from jax.experimental.pallas import tpu_sc as plsc

_NC = 2
_NS = 16
_NW = _NC * _NS
_PC = 32
_L = 16


@functools.lru_cache(maxsize=None)
def _build_sc_call(B, T, D, V, PMAX):
    PW = T // _NW
    NPC = PW // _PC
    NCH = NPC * B
    VPR = D // _L

    mesh = plsc.VectorSubcoreMesh(core_axis_name="c", subcore_axis_name="s")

    @functools.partial(
        pl.kernel,
        mesh=mesh,
        out_type=jax.ShapeDtypeStruct((B * T, D), jnp.float32),
        scratch_types=[
            pltpu.VMEM((NCH, _PC), jnp.int32),
            pltpu.VMEM((_PC, D), jnp.float32),
            pltpu.VMEM((2, _PC, D), jnp.float32),
            pltpu.SemaphoreType.DMA,
            pltpu.SemaphoreType.DMA,
            pltpu.SemaphoreType.DMA,
            pltpu.SemaphoreType.DMA,
        ],
    )
    def sc_embed(idx_hbm, emb_hbm, pos_hbm, out_hbm, idx_v, pos_v, tok_v,
                 gsem0, gsem1, wsem0, wsem1):
        gsems = [gsem0, gsem1]
        wsems = [wsem0, wsem1]
        wid = lax.axis_index("s") * _NC + lax.axis_index("c")
        pos0 = wid * PW

        pltpu.sync_copy(idx_hbm.at[wid], idx_v)

        def start_gather(k, buf):
            return pltpu.async_copy(emb_hbm.at[idx_v.at[k]], tok_v.at[buf],
                                    gsems[buf])

        def start_write(k, buf):
            pc, b = divmod(k, B)
            base = b * T + pos0 + pc * _PC
            return pltpu.async_copy(tok_v.at[buf], out_hbm.at[pl.ds(base, _PC)],
                                    wsems[buf])

        def add_pos(buf):
            @plsc.parallel_loop(0, _PC * VPR, unroll=8)
            def col_body(i):
                r = i // VPR
                sl = pl.ds((i - r * VPR) * _L, _L)
                tok_v[buf, r, sl] = tok_v[buf, r, sl] + pos_v[r, sl]

        g_handles = [None, None]
        w_handles = [None, None]
        g_handles[0] = start_gather(0, 0)
        for k in range(NCH):
            pc, b = divmod(k, B)
            buf = k % 2
            if b == 0:
                pltpu.sync_copy(pos_hbm.at[pl.ds(pos0 + pc * _PC, _PC)], pos_v)
            if k + 1 < NCH:
                nbuf = (k + 1) % 2
                if w_handles[nbuf] is not None:
                    w_handles[nbuf].wait()
                g_handles[nbuf] = start_gather(k + 1, nbuf)
            g_handles[buf].wait()
            add_pos(buf)
            w_handles[buf] = start_write(k, buf)
        w_handles[0].wait()
        w_handles[1].wait()

    return sc_embed


def kernel(token_ids, token_emb, pos_emb):
    B, T = token_ids.shape
    V, D = token_emb.shape
    PMAX = pos_emb.shape[0]
    PW = T // _NW
    NPC = PW // _PC

    ids = token_ids.astype(jnp.int32)
    idx = (ids.reshape(B, _NW, NPC, _PC)
              .transpose(1, 2, 0, 3)
              .reshape(_NW, NPC * B, _PC))

    sc_embed = _build_sc_call(B, T, D, V, PMAX)
    out_flat = sc_embed(idx, token_emb, pos_emb)
    return out_flat.reshape(B, T, D)
